# baseline (speedup 1.0000x reference)
import jax
import jax.numpy as jnp
from jax import lax
from jax.experimental import pallas as pl
from jax.experimental.pallas import tpu as pltpu
from jax.experimental.pallas import tpu_sc as plsc

N = 320000
D = 128
S = 10000

Q = 4
NC = N // Q

B = 2000
NBLK = NC // B
K = 128

NW = 32
RPW = NC // NW
CS = 125
NCH = RPW // CS
STRIPE = S // 16


def _score_body(h_ref, idx_ref, w1_ref, b1_ref, w2_ref, b2_ref,
                g_ref, den_ref, dacc_ref):
    i = pl.program_id(0)

    @pl.when(i == 0)
    def _init():
        dacc_ref[...] = jnp.zeros_like(dacc_ref)

    hb = h_ref[...]
    hidden = jnp.tanh(
        lax.dot(hb, w1_ref[...], preferred_element_type=jnp.float32)
        + b1_ref[...])
    s = lax.dot(hidden, w2_ref[...],
                preferred_element_type=jnp.float32) + b2_ref[...]
    e = jnp.exp(s)
    g_ref[...] = hb * e

    idx = idx_ref[0]
    lo0 = jnp.min(idx)
    hi = jnp.max(idx)

    def cond(lo):
        return lo <= hi

    def body(lo):
        lo_c = jnp.minimum(lo - lax.rem(lo, 8), S - K)
        kio = lax.broadcasted_iota(jnp.int32, (K, B), 0)
        idxb = jnp.broadcast_to(idx, (K, B))
        oh = (idxb == kio + lo_c) & (idxb >= lo)
        ohf = oh.astype(jnp.float32)
        dwin = lax.dot(ohf, e, preferred_element_type=jnp.float32)
        dacc_ref[pl.ds(lo_c, K), :] += dwin
        return lo_c + K

    lax.while_loop(cond, body, lo0)

    @pl.when(i == NBLK - 1)
    def _finish():
        den_ref[...] = dacc_ref[...]


def _stage_a(hq, idx3, W1, b1r, W2, b2r):
    return pl.pallas_call(
        _score_body,
        grid=(NBLK,),
        in_specs=[
            pl.BlockSpec((B, D), lambda i: (i, 0)),
            pl.BlockSpec((1, 1, B), lambda i: (i, 0, 0)),
            pl.BlockSpec((D, D), lambda i: (0, 0)),
            pl.BlockSpec((1, D), lambda i: (0, 0)),
            pl.BlockSpec((D, 1), lambda i: (0, 0)),
            pl.BlockSpec((1, 1), lambda i: (0, 0)),
        ],
        out_specs=[
            pl.BlockSpec((B, D), lambda i: (i, 0)),
            pl.BlockSpec((S, 1), lambda i: (0, 0)),
        ],
        out_shape=[
            jax.ShapeDtypeStruct((NC, D), jnp.float32),
            jax.ShapeDtypeStruct((S, 1), jnp.float32),
        ],
        scratch_shapes=[pltpu.VMEM((S, 1), jnp.float32)],
    )(hq, idx3, W1, b1r, W2, b2r)


def _sc_body(g_hbm, idx_hbm, zeros_hbm, npart_hbm,
             iv, gv0, gv1, num_shared, sem0, sem1):
    c = lax.axis_index("c")
    sid = lax.axis_index("s")
    w = sid * 2 + c
    base = w * RPW

    pltpu.sync_copy(zeros_hbm, num_shared.at[pl.ds(sid * STRIPE, STRIPE)])
    pltpu.sync_copy(idx_hbm.at[w], iv)
    plsc.subcore_barrier()

    def dma(jj, buf, sem):
        return pltpu.make_async_copy(
            g_hbm.at[pl.ds(base + jj * CS, CS)], buf, sem)

    dma(0, gv0, sem0).start()
    dma(1, gv1, sem1).start()

    @pl.loop(0, NCH, step=2)
    def _(j):
        for b, (buf, sem) in enumerate(((gv0, sem0), (gv1, sem1))):
            jj = j + b
            dma(jj, buf, sem).wait()
            pltpu.sync_copy(buf, num_shared.at[iv.at[jj]], add=True)
            nxt = jj + 2

            @pl.when(nxt < NCH)
            def _():
                dma(nxt, buf, sem).start()

    plsc.subcore_barrier()
    pltpu.sync_copy(num_shared.at[pl.ds(sid * STRIPE, STRIPE)],
                    npart_hbm.at[c, pl.ds(sid * STRIPE, STRIPE)])


def _stage_b(gq, idx3, zeros):
    mesh = plsc.VectorSubcoreMesh(core_axis_name="c", subcore_axis_name="s")
    cp = pltpu.CompilerParams(use_tc_tiling_on_sc=False)
    f = pl.kernel(
        _sc_body,
        out_type=jax.ShapeDtypeStruct((2, S, D), jnp.float32),
        mesh=mesh,
        scratch_types=[
            pltpu.VMEM((NCH, CS), jnp.int32),
            pltpu.VMEM((CS, D), jnp.float32),
            pltpu.VMEM((CS, D), jnp.float32),
            pltpu.VMEM_SHARED((S, D), jnp.float32),
            pltpu.SemaphoreType.DMA,
            pltpu.SemaphoreType.DMA,
        ],
        compiler_params=cp,
    )
    return f(gq, idx3, zeros)


SB = 1000


def _combine_body(n0_ref, n1_ref, n2_ref, n3_ref,
                  d0_ref, d1_ref, d2_ref, d3_ref, out_ref):
    num = (n0_ref[0] + n0_ref[1] + n1_ref[0] + n1_ref[1]
           + n2_ref[0] + n2_ref[1] + n3_ref[0] + n3_ref[1])
    den = d0_ref[...] + d1_ref[...] + d2_ref[...] + d3_ref[...]
    out_ref[...] = num / jnp.where(den > 0.0, den, 1.0)


def _stage_c(nparts, dens):
    npart_spec = pl.BlockSpec((2, SB, D), lambda i: (0, i, 0))
    den_spec = pl.BlockSpec((SB, 1), lambda i: (i, 0))
    return pl.pallas_call(
        _combine_body,
        grid=(S // SB,),
        in_specs=[npart_spec] * 4 + [den_spec] * 4,
        out_specs=pl.BlockSpec((SB, D), lambda i: (i, 0)),
        out_shape=jax.ShapeDtypeStruct((S, D), jnp.float32),
    )(*nparts, *dens)


@jax.jit
def kernel(h, batch_indices, W1, b1, W2, b2):
    b1r = b1.reshape(1, D)
    b2r = b2.reshape(1, 1)
    zeros = jnp.zeros((STRIPE, D), jnp.float32)
    hq = h.reshape(Q, NC, D)
    idxq = batch_indices.reshape(Q, NC)
    nparts, dens = [], []
    for q in range(Q):
        gq, dq = _stage_a(hq[q], idxq[q].reshape(NBLK, 1, B),
                          W1, b1r, W2, b2r)
        nparts.append(_stage_b(gq, idxq[q].reshape(NW, NCH, CS), zeros))
        dens.append(dq)
    return _stage_c(nparts, dens)


# speedup vs baseline: 1.3225x; 1.3225x over previous
import functools

import jax
import jax.numpy as jnp
from jax import lax
from jax.experimental import pallas as pl
from jax.experimental.pallas import tpu as pltpu
from jax.experimental.pallas import tpu_sc as plsc

N = 320000
D = 128
S = 10000

Q = 4
NC = N // Q

B = 2000
NBLK = NC // B
K = 128

NW = 32
RPW = NC // NW
CS = 125
NCH = RPW // CS
STRIPE = S // 16


def _score_body(h_ref, idx_ref, w1_ref, b1_ref, w2_ref, b2_ref,
                g_ref, den_ref, dacc_ref):
    i = pl.program_id(0)

    @pl.when(i == 0)
    def _init():
        dacc_ref[...] = jnp.zeros_like(dacc_ref)

    hb = h_ref[...]
    hidden = jnp.tanh(
        lax.dot(hb, w1_ref[...], preferred_element_type=jnp.float32)
        + b1_ref[...])
    s = lax.dot(hidden, w2_ref[...],
                preferred_element_type=jnp.float32) + b2_ref[...]
    e = jnp.exp(s)
    g_ref[...] = hb * e

    idx = idx_ref[0]
    lo0 = jnp.min(idx)
    hi = jnp.max(idx)

    def cond(lo):
        return lo <= hi

    def body(lo):
        lo_c = jnp.minimum(lo - lax.rem(lo, 8), S - K)
        kio = lax.broadcasted_iota(jnp.int32, (K, B), 0)
        idxb = jnp.broadcast_to(idx, (K, B))
        oh = (idxb == kio + lo_c) & (idxb >= lo)
        ohf = oh.astype(jnp.float32)
        dwin = lax.dot(ohf, e, preferred_element_type=jnp.float32)
        dacc_ref[pl.ds(lo_c, K), :] += dwin
        return lo_c + K

    lax.while_loop(cond, body, lo0)

    @pl.when(i == NBLK - 1)
    def _finish():
        den_ref[...] = dacc_ref[...]


def _stage_a(h, idx3, W1, b1r, W2, b2r, q):
    off = q * NBLK
    return pl.pallas_call(
        _score_body,
        grid=(NBLK,),
        in_specs=[
            pl.BlockSpec((B, D), lambda i: (off + i, 0)),
            pl.BlockSpec((1, 1, B), lambda i: (off + i, 0, 0)),
            pl.BlockSpec((D, D), lambda i: (0, 0)),
            pl.BlockSpec((1, D), lambda i: (0, 0)),
            pl.BlockSpec((D, 1), lambda i: (0, 0)),
            pl.BlockSpec((1, 1), lambda i: (0, 0)),
        ],
        out_specs=[
            pl.BlockSpec((B, D), lambda i: (i, 0)),
            pl.BlockSpec((S, 1), lambda i: (0, 0)),
        ],
        out_shape=[
            jax.ShapeDtypeStruct((NC, D), jnp.float32),
            jax.ShapeDtypeStruct((S, 1), jnp.float32),
        ],
        scratch_shapes=[pltpu.VMEM((S, 1), jnp.float32)],
    )(h, idx3, W1, b1r, W2, b2r)


def _sc_body(q, g_hbm, idx_hbm, zeros_hbm, npart_hbm,
             iv, gv0, gv1, num_shared, sem0, sem1):
    c = lax.axis_index("c")
    sid = lax.axis_index("s")
    w = sid * 2 + c
    base = w * RPW

    pltpu.sync_copy(zeros_hbm, num_shared.at[pl.ds(sid * STRIPE, STRIPE)])
    pltpu.sync_copy(idx_hbm.at[q * NW + w], iv)
    plsc.subcore_barrier()

    def dma(jj, buf, sem):
        return pltpu.make_async_copy(
            g_hbm.at[pl.ds(base + jj * CS, CS)], buf, sem)

    dma(0, gv0, sem0).start()
    dma(1, gv1, sem1).start()

    @pl.loop(0, NCH, step=2)
    def _(j):
        for b, (buf, sem) in enumerate(((gv0, sem0), (gv1, sem1))):
            jj = j + b
            dma(jj, buf, sem).wait()
            pltpu.sync_copy(buf, num_shared.at[iv.at[jj]], add=True)
            nxt = jj + 2

            @pl.when(nxt < NCH)
            def _():
                dma(nxt, buf, sem).start()

    plsc.subcore_barrier()
    pltpu.sync_copy(num_shared.at[pl.ds(sid * STRIPE, STRIPE)],
                    npart_hbm.at[c, pl.ds(sid * STRIPE, STRIPE)])


def _stage_b(gq, idx4, zeros, q):
    mesh = plsc.VectorSubcoreMesh(core_axis_name="c", subcore_axis_name="s")
    cp = pltpu.CompilerParams(use_tc_tiling_on_sc=False)
    f = pl.kernel(
        functools.partial(_sc_body, q),
        out_type=jax.ShapeDtypeStruct((2, S, D), jnp.float32),
        mesh=mesh,
        scratch_types=[
            pltpu.VMEM((NCH, CS), jnp.int32),
            pltpu.VMEM((CS, D), jnp.float32),
            pltpu.VMEM((CS, D), jnp.float32),
            pltpu.VMEM_SHARED((S, D), jnp.float32),
            pltpu.SemaphoreType.DMA,
            pltpu.SemaphoreType.DMA,
        ],
        compiler_params=cp,
    )
    return f(gq, idx4, zeros)


SB = 1000


def _combine_body(n0_ref, n1_ref, n2_ref, n3_ref,
                  d0_ref, d1_ref, d2_ref, d3_ref, out_ref):
    num = (n0_ref[0] + n0_ref[1] + n1_ref[0] + n1_ref[1]
           + n2_ref[0] + n2_ref[1] + n3_ref[0] + n3_ref[1])
    den = d0_ref[...] + d1_ref[...] + d2_ref[...] + d3_ref[...]
    out_ref[...] = num / jnp.where(den > 0.0, den, 1.0)


def _stage_c(nparts, dens):
    npart_spec = pl.BlockSpec((2, SB, D), lambda i: (0, i, 0))
    den_spec = pl.BlockSpec((SB, 1), lambda i: (i, 0))
    return pl.pallas_call(
        _combine_body,
        grid=(S // SB,),
        in_specs=[npart_spec] * 4 + [den_spec] * 4,
        out_specs=pl.BlockSpec((SB, D), lambda i: (i, 0)),
        out_shape=jax.ShapeDtypeStruct((S, D), jnp.float32),
    )(*nparts, *dens)


@jax.jit
def kernel(h, batch_indices, W1, b1, W2, b2):
    b1r = b1.reshape(1, D)
    b2r = b2.reshape(1, 1)
    zeros = jnp.zeros((STRIPE, D), jnp.float32)
    idx3a = batch_indices.reshape(N // B, 1, B)
    idx4 = batch_indices.reshape(Q * NW, NCH, CS)
    nparts, dens = [], []
    for q in range(Q):
        gq, dq = _stage_a(h, idx3a, W1, b1r, W2, b2r, q)
        nparts.append(_stage_b(gq, idx4, zeros, q))
        dens.append(dq)
    return _stage_c(nparts, dens)


# speedup vs baseline: 1.3323x; 1.0074x over previous
import functools

import jax
import jax.numpy as jnp
from jax import lax
from jax.experimental import pallas as pl
from jax.experimental.pallas import tpu as pltpu
from jax.experimental.pallas import tpu_sc as plsc

N = 320000
D = 128
S = 10000

Q = 4
NC = N // Q

B = 2000
NBLK = NC // B
K = 128

NW = 32
RPW = NC // NW
CS = 125
NCH = RPW // CS
STRIPE = S // 16


def _score_body(h_ref, idx_ref, w1_ref, b1_ref, w2_ref, b2_ref,
                g_ref, den_ref, dacc_ref):
    i = pl.program_id(0)

    @pl.when(i == 0)
    def _init():
        dacc_ref[...] = jnp.zeros_like(dacc_ref)

    hb = h_ref[...]
    hidden = jnp.tanh(
        lax.dot(hb, w1_ref[...], preferred_element_type=jnp.float32)
        + b1_ref[...])
    s = lax.dot(hidden, w2_ref[...],
                preferred_element_type=jnp.float32) + b2_ref[...]
    e = jnp.exp(s)
    g_ref[...] = hb * e

    idx = idx_ref[0]
    lo0 = jnp.min(idx)
    hi = jnp.max(idx)

    def cond(lo):
        return lo <= hi

    def body(lo):
        lo_c = jnp.minimum(lo - lax.rem(lo, 8), S - K)
        kio = lax.broadcasted_iota(jnp.int32, (K, B), 0)
        idxb = jnp.broadcast_to(idx, (K, B))
        oh = (idxb == kio + lo_c) & (idxb >= lo)
        ohf = oh.astype(jnp.float32)
        dwin = lax.dot(ohf, e, preferred_element_type=jnp.float32)
        dacc_ref[pl.ds(lo_c, K), :] += dwin
        return lo_c + K

    lax.while_loop(cond, body, lo0)

    @pl.when(i == NBLK - 1)
    def _finish():
        den_ref[...] = dacc_ref[...]


def _stage_a(h, idx3, W1, b1r, W2, b2r, q):
    off = q * NBLK
    return pl.pallas_call(
        _score_body,
        grid=(NBLK,),
        in_specs=[
            pl.BlockSpec((B, D), lambda i: (off + i, 0)),
            pl.BlockSpec((1, 1, B), lambda i: (off + i, 0, 0)),
            pl.BlockSpec((D, D), lambda i: (0, 0)),
            pl.BlockSpec((1, D), lambda i: (0, 0)),
            pl.BlockSpec((D, 1), lambda i: (0, 0)),
            pl.BlockSpec((1, 1), lambda i: (0, 0)),
        ],
        out_specs=[
            pl.BlockSpec((B, D), lambda i: (i, 0)),
            pl.BlockSpec((S, 1), lambda i: (0, 0)),
        ],
        out_shape=[
            jax.ShapeDtypeStruct((NC, D), jnp.float32),
            jax.ShapeDtypeStruct((S, 1), jnp.float32),
        ],
        scratch_shapes=[pltpu.VMEM((S, 1), jnp.float32)],
    )(h, idx3, W1, b1r, W2, b2r)


def _sc_body(q, g_hbm, idx_hbm, zeros_hbm, npart_hbm,
             iv, gv0, gv1, num_shared, sem0, sem1):
    c = lax.axis_index("c")
    sid = lax.axis_index("s")
    w = sid * 2 + c
    base = w * RPW

    pltpu.sync_copy(zeros_hbm, num_shared.at[pl.ds(sid * STRIPE, STRIPE)])
    pltpu.sync_copy(idx_hbm.at[w], iv)
    plsc.subcore_barrier()

    def dma(jj, buf, sem):
        return pltpu.make_async_copy(
            g_hbm.at[pl.ds(base + jj * CS, CS)], buf, sem)

    dma(0, gv0, sem0).start()
    dma(1, gv1, sem1).start()

    @pl.loop(0, NCH, step=2)
    def _(j):
        for b, (buf, sem) in enumerate(((gv0, sem0), (gv1, sem1))):
            jj = j + b
            dma(jj, buf, sem).wait()
            pltpu.sync_copy(buf, num_shared.at[iv.at[jj]], add=True)
            nxt = jj + 2

            @pl.when(nxt < NCH)
            def _():
                dma(nxt, buf, sem).start()

    plsc.subcore_barrier()
    pltpu.sync_copy(num_shared.at[pl.ds(sid * STRIPE, STRIPE)],
                    npart_hbm.at[c, pl.ds(sid * STRIPE, STRIPE)])


def _stage_b(gq, idx4, zeros, q):
    mesh = plsc.VectorSubcoreMesh(core_axis_name="c", subcore_axis_name="s")
    cp = pltpu.CompilerParams(use_tc_tiling_on_sc=False)
    f = pl.kernel(
        functools.partial(_sc_body, q),
        out_type=jax.ShapeDtypeStruct((2, S, D), jnp.float32),
        mesh=mesh,
        scratch_types=[
            pltpu.VMEM((NCH, CS), jnp.int32),
            pltpu.VMEM((CS, D), jnp.float32),
            pltpu.VMEM((CS, D), jnp.float32),
            pltpu.VMEM_SHARED((S, D), jnp.float32),
            pltpu.SemaphoreType.DMA,
            pltpu.SemaphoreType.DMA,
        ],
        compiler_params=cp,
    )
    return f(gq, idx4, zeros)


SB = 1000


def _combine_body(n0_ref, n1_ref, n2_ref, n3_ref,
                  d0_ref, d1_ref, d2_ref, d3_ref, out_ref):
    num = (n0_ref[0] + n0_ref[1] + n1_ref[0] + n1_ref[1]
           + n2_ref[0] + n2_ref[1] + n3_ref[0] + n3_ref[1])
    den = d0_ref[...] + d1_ref[...] + d2_ref[...] + d3_ref[...]
    out_ref[...] = num / jnp.where(den > 0.0, den, 1.0)


def _stage_c(nparts, dens):
    npart_spec = pl.BlockSpec((2, SB, D), lambda i: (0, i, 0))
    den_spec = pl.BlockSpec((SB, 1), lambda i: (i, 0))
    return pl.pallas_call(
        _combine_body,
        grid=(S // SB,),
        in_specs=[npart_spec] * 4 + [den_spec] * 4,
        out_specs=pl.BlockSpec((SB, D), lambda i: (i, 0)),
        out_shape=jax.ShapeDtypeStruct((S, D), jnp.float32),
    )(*nparts, *dens)


@jax.jit
def kernel(h, batch_indices, W1, b1, W2, b2):
    b1r = b1.reshape(1, D)
    b2r = b2.reshape(1, 1)
    zeros = jnp.zeros((STRIPE, D), jnp.float32)
    idx3a = batch_indices.reshape(N // B, 1, B)
    idx4 = batch_indices.reshape(Q, NW, NCH, CS)
    nparts, dens = [], []
    for q in range(Q):
        gq, dq = _stage_a(h, idx3a, W1, b1r, W2, b2r, q)
        nparts.append(_stage_b(gq, idx4[q], zeros, q))
        dens.append(dq)
    return _stage_c(nparts, dens)


# speedup vs baseline: 1.5046x; 1.1293x over previous
import functools

import jax
import jax.numpy as jnp
from jax import lax
from jax.experimental import pallas as pl
from jax.experimental.pallas import tpu as pltpu
from jax.experimental.pallas import tpu_sc as plsc

N = 320000
D = 128
S = 10000

Q = 4
NC = N // Q

B = 3200
NBLK = NC // B
K = 128

NW = 32
RPW = NC // NW
CS = 125
NCH = RPW // CS
STRIPE = S // 16


def _score_body(h_ref, idx_ref, w1_ref, b1_ref, w2_ref, b2_ref,
                g_ref, den_ref, dacc_ref):
    i = pl.program_id(0)

    @pl.when(i == 0)
    def _init():
        dacc_ref[...] = jnp.zeros_like(dacc_ref)

    hb = h_ref[...]
    hidden = jnp.tanh(
        lax.dot(hb, w1_ref[...], preferred_element_type=jnp.float32)
        + b1_ref[...])
    s = lax.dot(hidden, w2_ref[...],
                preferred_element_type=jnp.float32) + b2_ref[...]
    e = jnp.exp(s)
    g_ref[...] = hb * e

    idx = idx_ref[0]
    lo0 = jnp.min(idx)
    hi = jnp.max(idx)

    def cond(lo):
        return lo <= hi

    def body(lo):
        lo_c = jnp.minimum(lo - lax.rem(lo, 8), S - K)
        kio = lax.broadcasted_iota(jnp.int32, (K, B), 0)
        idxb = jnp.broadcast_to(idx, (K, B))
        oh = (idxb == kio + lo_c) & (idxb >= lo)
        ohf = oh.astype(jnp.float32)
        dwin = lax.dot(ohf, e, preferred_element_type=jnp.float32)
        dacc_ref[pl.ds(lo_c, K), :] += dwin
        return lo_c + K

    lax.while_loop(cond, body, lo0)

    @pl.when(i == NBLK - 1)
    def _finish():
        den_ref[...] = dacc_ref[...]


def _stage_a(h, idx3, W1, b1r, W2, b2r, q):
    off = q * NBLK
    return pl.pallas_call(
        _score_body,
        grid=(NBLK,),
        in_specs=[
            pl.BlockSpec((B, D), lambda i: (off + i, 0)),
            pl.BlockSpec((1, 1, B), lambda i: (off + i, 0, 0)),
            pl.BlockSpec((D, D), lambda i: (0, 0)),
            pl.BlockSpec((1, D), lambda i: (0, 0)),
            pl.BlockSpec((D, 1), lambda i: (0, 0)),
            pl.BlockSpec((1, 1), lambda i: (0, 0)),
        ],
        out_specs=[
            pl.BlockSpec((B, D), lambda i: (i, 0)),
            pl.BlockSpec((S, 1), lambda i: (0, 0)),
        ],
        out_shape=[
            jax.ShapeDtypeStruct((NC, D), jnp.float32),
            jax.ShapeDtypeStruct((S, 1), jnp.float32),
        ],
        scratch_shapes=[pltpu.VMEM((S, 1), jnp.float32)],
    )(h, idx3, W1, b1r, W2, b2r)


def _sc_body(q, g_hbm, idx_hbm, zeros_hbm, npart_hbm,
             iv, gv0, gv1, num_shared, sem0, sem1):
    c = lax.axis_index("c")
    sid = lax.axis_index("s")
    w = sid * 2 + c
    base = w * RPW

    pltpu.sync_copy(zeros_hbm, num_shared.at[pl.ds(sid * STRIPE, STRIPE)])
    pltpu.sync_copy(idx_hbm.at[w], iv)
    plsc.subcore_barrier()

    def dma(jj, buf, sem):
        return pltpu.make_async_copy(
            g_hbm.at[pl.ds(base + jj * CS, CS)], buf, sem)

    dma(0, gv0, sem0).start()
    dma(1, gv1, sem1).start()

    @pl.loop(0, NCH, step=2)
    def _(j):
        for b, (buf, sem) in enumerate(((gv0, sem0), (gv1, sem1))):
            jj = j + b
            dma(jj, buf, sem).wait()
            pltpu.sync_copy(buf, num_shared.at[iv.at[jj]], add=True)
            nxt = jj + 2

            @pl.when(nxt < NCH)
            def _():
                dma(nxt, buf, sem).start()

    plsc.subcore_barrier()
    pltpu.sync_copy(num_shared.at[pl.ds(sid * STRIPE, STRIPE)],
                    npart_hbm.at[c, pl.ds(sid * STRIPE, STRIPE)])


def _stage_b(gq, idx4, zeros, q):
    mesh = plsc.VectorSubcoreMesh(core_axis_name="c", subcore_axis_name="s")
    cp = pltpu.CompilerParams(use_tc_tiling_on_sc=False)
    f = pl.kernel(
        functools.partial(_sc_body, q),
        out_type=jax.ShapeDtypeStruct((2, S, D), jnp.float32),
        mesh=mesh,
        scratch_types=[
            pltpu.VMEM((NCH, CS), jnp.int32),
            pltpu.VMEM((CS, D), jnp.float32),
            pltpu.VMEM((CS, D), jnp.float32),
            pltpu.VMEM_SHARED((S, D), jnp.float32),
            pltpu.SemaphoreType.DMA,
            pltpu.SemaphoreType.DMA,
        ],
        compiler_params=cp,
    )
    return f(gq, idx4, zeros)


SB = 1000


def _combine_body(n0_ref, n1_ref, n2_ref, n3_ref,
                  d0_ref, d1_ref, d2_ref, d3_ref, out_ref):
    num = (n0_ref[0] + n0_ref[1] + n1_ref[0] + n1_ref[1]
           + n2_ref[0] + n2_ref[1] + n3_ref[0] + n3_ref[1])
    den = d0_ref[...] + d1_ref[...] + d2_ref[...] + d3_ref[...]
    out_ref[...] = num / jnp.where(den > 0.0, den, 1.0)


def _stage_c(nparts, dens):
    npart_spec = pl.BlockSpec((2, SB, D), lambda i: (0, i, 0))
    den_spec = pl.BlockSpec((SB, 1), lambda i: (i, 0))
    return pl.pallas_call(
        _combine_body,
        grid=(S // SB,),
        in_specs=[npart_spec] * 4 + [den_spec] * 4,
        out_specs=pl.BlockSpec((SB, D), lambda i: (i, 0)),
        out_shape=jax.ShapeDtypeStruct((S, D), jnp.float32),
    )(*nparts, *dens)


@jax.jit
def kernel(h, batch_indices, W1, b1, W2, b2):
    b1r = b1.reshape(1, D)
    b2r = b2.reshape(1, 1)
    zeros = jnp.zeros((STRIPE, D), jnp.float32)
    idx3a = batch_indices.reshape(N // B, 1, B)
    idx4 = batch_indices.reshape(Q, NW, NCH, CS)
    nparts, dens = [], []
    for q in range(Q):
        gq, dq = _stage_a(h, idx3a, W1, b1r, W2, b2r, q)
        nparts.append(_stage_b(gq, idx4[q], zeros, q))
        dens.append(dq)
    return _stage_c(nparts, dens)


# speedup vs baseline: 1.5236x; 1.0126x over previous
import functools

import jax
import jax.numpy as jnp
from jax import lax
from jax.experimental import pallas as pl
from jax.experimental.pallas import tpu as pltpu
from jax.experimental.pallas import tpu_sc as plsc

N = 320000
D = 128
S = 10000

Q = 4
NC = N // Q

B = 4000
NBLK = NC // B
K = 128

NW = 32
RPW = NC // NW
CS = 125
NCH = RPW // CS
STRIPE = S // 16


def _score_body(h_ref, idx_ref, w1_ref, b1_ref, w2_ref, b2_ref,
                g_ref, den_ref, dacc_ref):
    i = pl.program_id(0)

    @pl.when(i == 0)
    def _init():
        dacc_ref[...] = jnp.zeros_like(dacc_ref)

    hb = h_ref[...]
    hidden = jnp.tanh(
        lax.dot(hb, w1_ref[...], preferred_element_type=jnp.float32)
        + b1_ref[...])
    s = lax.dot(hidden, w2_ref[...],
                preferred_element_type=jnp.float32) + b2_ref[...]
    e = jnp.exp(s)
    g_ref[...] = hb * e

    idx = idx_ref[0]
    lo0 = jnp.min(idx)
    hi = jnp.max(idx)

    def cond(lo):
        return lo <= hi

    def body(lo):
        lo_c = jnp.minimum(lo - lax.rem(lo, 8), S - K)
        kio = lax.broadcasted_iota(jnp.int32, (K, B), 0)
        idxb = jnp.broadcast_to(idx, (K, B))
        oh = (idxb == kio + lo_c) & (idxb >= lo)
        ohf = oh.astype(jnp.float32)
        dwin = lax.dot(ohf, e, preferred_element_type=jnp.float32)
        dacc_ref[pl.ds(lo_c, K), :] += dwin
        return lo_c + K

    lax.while_loop(cond, body, lo0)

    @pl.when(i == NBLK - 1)
    def _finish():
        den_ref[...] = dacc_ref[...]


def _stage_a(h, idx3, W1, b1r, W2, b2r, q):
    off = q * NBLK
    return pl.pallas_call(
        _score_body,
        grid=(NBLK,),
        in_specs=[
            pl.BlockSpec((B, D), lambda i: (off + i, 0)),
            pl.BlockSpec((1, 1, B), lambda i: (off + i, 0, 0)),
            pl.BlockSpec((D, D), lambda i: (0, 0)),
            pl.BlockSpec((1, D), lambda i: (0, 0)),
            pl.BlockSpec((D, 1), lambda i: (0, 0)),
            pl.BlockSpec((1, 1), lambda i: (0, 0)),
        ],
        out_specs=[
            pl.BlockSpec((B, D), lambda i: (i, 0)),
            pl.BlockSpec((S, 1), lambda i: (0, 0)),
        ],
        out_shape=[
            jax.ShapeDtypeStruct((NC, D), jnp.float32),
            jax.ShapeDtypeStruct((S, 1), jnp.float32),
        ],
        scratch_shapes=[pltpu.VMEM((S, 1), jnp.float32)],
    )(h, idx3, W1, b1r, W2, b2r)


def _sc_body(q, g_hbm, idx_hbm, zeros_hbm, npart_hbm,
             iv, gv0, gv1, num_shared, sem0, sem1):
    c = lax.axis_index("c")
    sid = lax.axis_index("s")
    w = sid * 2 + c
    base = w * RPW

    pltpu.sync_copy(zeros_hbm, num_shared.at[pl.ds(sid * STRIPE, STRIPE)])
    pltpu.sync_copy(idx_hbm.at[w], iv)
    plsc.subcore_barrier()

    def dma(jj, buf, sem):
        return pltpu.make_async_copy(
            g_hbm.at[pl.ds(base + jj * CS, CS)], buf, sem)

    dma(0, gv0, sem0).start()
    dma(1, gv1, sem1).start()

    @pl.loop(0, NCH, step=2)
    def _(j):
        for b, (buf, sem) in enumerate(((gv0, sem0), (gv1, sem1))):
            jj = j + b
            dma(jj, buf, sem).wait()
            pltpu.sync_copy(buf, num_shared.at[iv.at[jj]], add=True)
            nxt = jj + 2

            @pl.when(nxt < NCH)
            def _():
                dma(nxt, buf, sem).start()

    plsc.subcore_barrier()
    pltpu.sync_copy(num_shared.at[pl.ds(sid * STRIPE, STRIPE)],
                    npart_hbm.at[c, pl.ds(sid * STRIPE, STRIPE)])


def _stage_b(gq, idx4, zeros, q):
    mesh = plsc.VectorSubcoreMesh(core_axis_name="c", subcore_axis_name="s")
    cp = pltpu.CompilerParams(use_tc_tiling_on_sc=False)
    f = pl.kernel(
        functools.partial(_sc_body, q),
        out_type=jax.ShapeDtypeStruct((2, S, D), jnp.float32),
        mesh=mesh,
        scratch_types=[
            pltpu.VMEM((NCH, CS), jnp.int32),
            pltpu.VMEM((CS, D), jnp.float32),
            pltpu.VMEM((CS, D), jnp.float32),
            pltpu.VMEM_SHARED((S, D), jnp.float32),
            pltpu.SemaphoreType.DMA,
            pltpu.SemaphoreType.DMA,
        ],
        compiler_params=cp,
    )
    return f(gq, idx4, zeros)


SB = 1000


def _combine_body(n0_ref, n1_ref, n2_ref, n3_ref,
                  d0_ref, d1_ref, d2_ref, d3_ref, out_ref):
    num = (n0_ref[0] + n0_ref[1] + n1_ref[0] + n1_ref[1]
           + n2_ref[0] + n2_ref[1] + n3_ref[0] + n3_ref[1])
    den = d0_ref[...] + d1_ref[...] + d2_ref[...] + d3_ref[...]
    out_ref[...] = num / jnp.where(den > 0.0, den, 1.0)


def _stage_c(nparts, dens):
    npart_spec = pl.BlockSpec((2, SB, D), lambda i: (0, i, 0))
    den_spec = pl.BlockSpec((SB, 1), lambda i: (i, 0))
    return pl.pallas_call(
        _combine_body,
        grid=(S // SB,),
        in_specs=[npart_spec] * 4 + [den_spec] * 4,
        out_specs=pl.BlockSpec((SB, D), lambda i: (i, 0)),
        out_shape=jax.ShapeDtypeStruct((S, D), jnp.float32),
    )(*nparts, *dens)


@jax.jit
def kernel(h, batch_indices, W1, b1, W2, b2):
    b1r = b1.reshape(1, D)
    b2r = b2.reshape(1, 1)
    zeros = jnp.zeros((STRIPE, D), jnp.float32)
    idx3a = batch_indices.reshape(N // B, 1, B)
    idx4 = batch_indices.reshape(Q, NW, NCH, CS)
    nparts, dens = [], []
    for q in range(Q):
        gq, dq = _stage_a(h, idx3a, W1, b1r, W2, b2r, q)
        nparts.append(_stage_b(gq, idx4[q], zeros, q))
        dens.append(dq)
    return _stage_c(nparts, dens)
